# Initial kernel scaffold; baseline (speedup 1.0000x reference)
import jax
import jax.numpy as jnp
from jax.experimental import pallas as pl


def kernel(x, table, W, b):
    raise NotImplementedError("write your pallas kernel here")


# speedup vs baseline: 10.9127x; 10.9127x over previous
import functools

import jax
import jax.numpy as jnp
from jax import lax
from jax.experimental import pallas as pl
from jax.experimental.pallas import tpu as pltpu
from jax.experimental.pallas import tpu_sc as plsc

B, H, D = 16384, 200, 32
NC, NS = 2, 16
NW = NC * NS
PER_W = B // NW
KR = 8
CHUNK = KR * 128
NCH = PER_W * H // CHUNK
ACC_ROWS = NS * PER_W


def _sc_gather_sum(x_r, dst_r, table):
  mesh = plsc.VectorSubcoreMesh(core_axis_name="c", subcore_axis_name="s")

  @functools.partial(
      pl.kernel,
      out_type=jax.ShapeDtypeStruct((B, D), jnp.float32),
      mesh=mesh,
      scratch_types=[
          pltpu.VMEM((KR, 128), jnp.int32),
          pltpu.VMEM((KR, 128), jnp.int32),
          pltpu.VMEM((KR, 128, D), jnp.float32),
          pltpu.VMEM((PER_W, D), jnp.float32),
          pltpu.VMEM_SHARED((ACC_ROWS, D), jnp.float32),
          pltpu.SemaphoreType.DMA,
      ],
      compiler_params=pltpu.CompilerParams(use_tc_tiling_on_sc=False),
  )
  def k(x_hbm, dst_hbm, tab_hbm, out_hbm,
        idx_v, dst_v, rows_v, stage_v, acc_sh, gsem):
    cid = lax.axis_index("c")
    sid = lax.axis_index("s")
    wid = cid * NS + sid

    zero = jnp.zeros((16,), jnp.float32)

    def _zero_row(i, carry):
      stage_v[i, pl.ds(0, 16)] = zero
      stage_v[i, pl.ds(16, 16)] = zero
      return carry

    lax.fori_loop(0, PER_W, _zero_row, 0)
    pltpu.sync_copy(stage_v, acc_sh.at[pl.ds(sid * PER_W, PER_W)])

    def _chunk(ci, carry):
      pltpu.sync_copy(x_hbm.at[wid, ci], idx_v)
      pltpu.sync_copy(dst_hbm.at[wid, ci], dst_v)
      descs = [
          pltpu.async_copy(tab_hbm.at[idx_v.at[j]], rows_v.at[j], gsem)
          for j in range(KR)
      ]
      for d in descs:
        d.wait()
      for j in range(KR):
        pltpu.sync_copy(rows_v.at[j], acc_sh.at[dst_v.at[j]], add=True)
      return carry

    lax.fori_loop(0, NCH, _chunk, 0)

    pltpu.sync_copy(acc_sh.at[pl.ds(sid * PER_W, PER_W)], stage_v)
    pltpu.sync_copy(stage_v, out_hbm.at[pl.ds(wid * PER_W, PER_W)])

  return k(x_r, dst_r, table)


def _tc_body(s_ref, wt_ref, b_ref, o_ref):
  o_ref[...] = (
      jnp.dot(s_ref[...], wt_ref[...], preferred_element_type=jnp.float32)
      * (1.0 / H)
      + b_ref[...]
  )


def _tc_project(sums, wt, b2):
  blk = 2048
  return pl.pallas_call(
      _tc_body,
      grid=(B // blk,),
      in_specs=[
          pl.BlockSpec((blk, D), lambda i: (i, 0)),
          pl.BlockSpec((D, D), lambda i: (0, 0)),
          pl.BlockSpec((1, D), lambda i: (0, 0)),
      ],
      out_specs=pl.BlockSpec((blk, D), lambda i: (i, 0)),
      out_shape=jax.ShapeDtypeStruct((B, D), jnp.float32),
  )(sums, wt, b2)


def kernel(x, table, W, b):
  x_r = x.astype(jnp.int32).reshape(NW, NCH, KR, 128)
  dst_r = ((jnp.arange(B * H, dtype=jnp.int32) // H) % ACC_ROWS).reshape(
      NW, NCH, KR, 128)
  sums = _sc_gather_sum(x_r, dst_r, table)
  return _tc_project(sums, W.T, b.reshape(1, D))


# speedup vs baseline: 12.7296x; 1.1665x over previous
import functools

import jax
import jax.numpy as jnp
from jax import lax
from jax.experimental import pallas as pl
from jax.experimental.pallas import tpu as pltpu
from jax.experimental.pallas import tpu_sc as plsc

B, H, D = 16384, 200, 32
NC, NS = 2, 16
NW = NC * NS
PER_W = B // NW
KR = 8
CHUNK = KR * 128
NCH = PER_W * H // CHUNK
NB = 3
ACC_ROWS = NS * PER_W


def _sc_gather_sum(x_r, dst_r, table):
  mesh = plsc.VectorSubcoreMesh(core_axis_name="c", subcore_axis_name="s")

  @functools.partial(
      pl.kernel,
      out_type=jax.ShapeDtypeStruct((B, D), jnp.float32),
      mesh=mesh,
      scratch_types=[
          pltpu.VMEM((NB, KR, 128), jnp.int32),
          pltpu.VMEM((NB, KR, 128), jnp.int32),
          pltpu.VMEM((NB, CHUNK, D), jnp.float32),
          pltpu.VMEM_SHARED((ACC_ROWS, D), jnp.float32),
          pltpu.SemaphoreType.DMA,
          pltpu.SemaphoreType.DMA,
          pltpu.SemaphoreType.DMA,
      ],
      compiler_params=pltpu.CompilerParams(use_tc_tiling_on_sc=False),
  )
  def k(x_hbm, dst_hbm, tab_hbm, out_hbm,
        idx_v, dst_v, rows_v, acc_sh, isem, gsem, ssem):
    cid = lax.axis_index("c")
    sid = lax.axis_index("s")
    wid = cid * NS + sid

    pltpu.async_copy(x_hbm.at[wid, 0], idx_v.at[0], isem)
    pltpu.async_copy(dst_hbm.at[wid, 0], dst_v.at[0], isem)

    zero = jnp.zeros((16,), jnp.float32)

    def _zero_row(i, carry):
      rows_v[0, i, pl.ds(0, 16)] = zero
      rows_v[0, i, pl.ds(16, 16)] = zero
      return carry

    lax.fori_loop(0, PER_W, _zero_row, 0)
    pltpu.sync_copy(rows_v.at[0, pl.ds(0, PER_W)],
                    acc_sh.at[pl.ds(sid * PER_W, PER_W)])

    def _chunk(ci, carry):
      slot = lax.rem(ci, NB)
      nslot = lax.rem(ci + 1, NB)
      pltpu.make_async_copy(x_hbm.at[wid, ci], idx_v.at[slot], isem).wait()
      pltpu.make_async_copy(dst_hbm.at[wid, ci], dst_v.at[slot], isem).wait()

      @pl.when(ci >= 2)
      def _():
        pltpu.make_async_copy(
            out_hbm.at[pl.ds(0, CHUNK)], rows_v.at[nslot], ssem).wait()

      descs = [
          pltpu.async_copy(
              tab_hbm.at[idx_v.at[slot, j]],
              rows_v.at[slot, pl.ds(j * 128, 128)], gsem)
          for j in range(KR)
      ]

      @pl.when(ci + 1 < NCH)
      def _():
        pltpu.async_copy(x_hbm.at[wid, ci + 1], idx_v.at[nslot], isem)
        pltpu.async_copy(dst_hbm.at[wid, ci + 1], dst_v.at[nslot], isem)

      for d in descs:
        d.wait()
      for j in range(KR):
        pltpu.async_copy(
            rows_v.at[slot, pl.ds(j * 128, 128)],
            acc_sh.at[dst_v.at[slot, j]], ssem, add=True)
      return carry

    lax.fori_loop(0, NCH, _chunk, 0)

    pltpu.make_async_copy(out_hbm.at[pl.ds(0, CHUNK)], rows_v.at[0], ssem).wait()
    pltpu.make_async_copy(out_hbm.at[pl.ds(0, CHUNK)], rows_v.at[1], ssem).wait()

    pltpu.sync_copy(acc_sh.at[pl.ds(sid * PER_W, PER_W)],
                    rows_v.at[0, pl.ds(0, PER_W)])
    pltpu.sync_copy(rows_v.at[0, pl.ds(0, PER_W)],
                    out_hbm.at[pl.ds(wid * PER_W, PER_W)])

  return k(x_r, dst_r, table)


def _tc_body(s_ref, wt_ref, b_ref, o_ref):
  o_ref[...] = (
      jnp.dot(s_ref[...], wt_ref[...], preferred_element_type=jnp.float32)
      * (1.0 / H)
      + b_ref[...]
  )


def _tc_project(sums, wt, b2):
  blk = 2048
  return pl.pallas_call(
      _tc_body,
      grid=(B // blk,),
      in_specs=[
          pl.BlockSpec((blk, D), lambda i: (i, 0)),
          pl.BlockSpec((D, D), lambda i: (0, 0)),
          pl.BlockSpec((1, D), lambda i: (0, 0)),
      ],
      out_specs=pl.BlockSpec((blk, D), lambda i: (i, 0)),
      out_shape=jax.ShapeDtypeStruct((B, D), jnp.float32),
  )(sums, wt, b2)


def kernel(x, table, W, b):
  x_r = x.astype(jnp.int32).reshape(NW, NCH, KR, 128)
  dst_r = ((jnp.arange(B * H, dtype=jnp.int32) // H) % ACC_ROWS).reshape(
      NW, NCH, KR, 128)
  sums = _sc_gather_sum(x_r, dst_r, table)
  return _tc_project(sums, W.T, b.reshape(1, D))
